# Initial kernel scaffold; baseline (speedup 1.0000x reference)
import jax
import jax.numpy as jnp
from jax.experimental import pallas as pl


def kernel(inputs):
    raise NotImplementedError("write your pallas kernel here")


# speedup vs baseline: 1.5107x; 1.5107x over previous
import jax
import jax.numpy as jnp
from jax.experimental import pallas as pl

_B = 32
_A = 3
_ATTR = 85
_GW = 76
_S = _GW * _GW
_C = _S
_NC = 1
_STRIDE = 8.0
_ANCH_W = (116.0, 156.0, 373.0)
_ANCH_H = (90.0, 198.0, 326.0)


def _decode_block(x_ref, o_ref):
    a = pl.program_id(1)
    c = pl.program_id(2)
    v = x_ref[0, 0]
    aw = jnp.where(a == 0, _ANCH_W[0], jnp.where(a == 1, _ANCH_W[1], _ANCH_W[2]))
    ah = jnp.where(a == 0, _ANCH_H[0], jnp.where(a == 1, _ANCH_H[1], _ANCH_H[2]))
    sig = jax.nn.sigmoid(v)
    wh = jnp.exp(v[2:4]) * jnp.stack([aw, ah])[:, None]
    t = jnp.concatenate([sig[0:2] * _STRIDE, wh, sig[4:]], axis=0)
    tt = t.T
    col = jax.lax.broadcasted_iota(jnp.int32, (_C, _ATTR), 1)
    s_abs = jax.lax.broadcasted_iota(jnp.int32, (_C, _ATTR), 0) + c * _C
    gx = (s_abs % _GW).astype(jnp.float32) * _STRIDE
    gy = (s_abs // _GW).astype(jnp.float32) * _STRIDE
    offs = jnp.where(col == 0, gx, jnp.where(col == 1, gy, 0.0))
    o_ref[0, 0] = tt + offs


def kernel(inputs):
    x4 = inputs.reshape(_B, _A, _ATTR, _S)
    out4 = pl.pallas_call(
        _decode_block,
        grid=(_B, _A, _NC),
        in_specs=[pl.BlockSpec((1, 1, _ATTR, _C), lambda b, a, c: (b, a, 0, c))],
        out_specs=pl.BlockSpec((1, 1, _C, _ATTR), lambda b, a, c: (b, a, c, 0)),
        out_shape=jax.ShapeDtypeStruct((_B, _A, _S, _ATTR), jnp.float32),
    )(x4)
    return out4.reshape(_B, _A * _S, _ATTR)


# speedup vs baseline: 1.5887x; 1.0516x over previous
import jax
import jax.numpy as jnp
from jax.experimental import pallas as pl

_B = 32
_A = 3
_ATTR = 85
_GW = 76
_S = _GW * _GW
_C = _S
_NC = 1
_STRIDE = 8.0
_ANCH_W = (116.0, 156.0, 373.0)
_ANCH_H = (90.0, 198.0, 326.0)


def _decode_block(x_ref, o_ref):
    a = pl.program_id(1)
    v = x_ref[0, 0]
    aw = jnp.where(a == 0, _ANCH_W[0], jnp.where(a == 1, _ANCH_W[1], _ANCH_W[2]))
    ah = jnp.where(a == 0, _ANCH_H[0], jnp.where(a == 1, _ANCH_H[1], _ANCH_H[2]))
    sig = jax.nn.sigmoid(v)
    s_iota = jax.lax.broadcasted_iota(jnp.int32, (1, _C), 1)
    gx = (s_iota % _GW).astype(jnp.float32)
    gy = (s_iota // _GW).astype(jnp.float32)
    row0 = (sig[0:1] + gx) * _STRIDE
    row1 = (sig[1:2] + gy) * _STRIDE
    wh = jnp.exp(v[2:4]) * jnp.stack([aw, ah])[:, None]
    t = jnp.concatenate([row0, row1, wh, sig[4:]], axis=0)
    o_ref[0, 0] = t.T


def kernel(inputs):
    x4 = inputs.reshape(_B, _A, _ATTR, _S)
    out4 = pl.pallas_call(
        _decode_block,
        grid=(_B, _A, _NC),
        in_specs=[pl.BlockSpec((1, 1, _ATTR, _C), lambda b, a, c: (b, a, 0, c))],
        out_specs=pl.BlockSpec((1, 1, _C, _ATTR), lambda b, a, c: (b, a, c, 0)),
        out_shape=jax.ShapeDtypeStruct((_B, _A, _S, _ATTR), jnp.float32),
    )(x4)
    return out4.reshape(_B, _A * _S, _ATTR)
